# baseline (device time: 105785 ns/iter reference)
import jax
import jax.numpy as jnp
from jax import lax
from jax.experimental import pallas as pl
from jax.experimental.pallas import tpu as pltpu

N_DEV = 32
R_STEPS = 16
L_STEPS = 15


def _build_ring():
    coords = [(x, y, z) for z in range(4) for y in range(4) for x in range(2)]
    logical = {}
    l = 0
    for z in range(4):
        plane = sorted(c for c in coords if c[2] == z)
        for yi, y in enumerate(sorted({c[1] for c in plane})):
            for c in sorted((c for c in plane if c[1] == y),
                            reverse=bool(yi % 2)):
                logical[c] = l
                l += 1
    cyc = []
    for y in range(4):
        zs = range(4) if y % 2 == 0 else range(3, -1, -1)
        cyc += [(0, y, z) for z in zs]
    for y in range(3, -1, -1):
        zs = range(4) if y % 2 == 1 else range(3, -1, -1)
        cyc += [(1, y, z) for z in zs]
    assert len(set(cyc)) == N_DEV
    for a, b in zip(cyc, cyc[1:] + cyc[:1]):
        assert sum(abs(i - j) for i, j in zip(a, b)) == 1, (a, b)
    return [logical[c] for c in cyc]


_RING = _build_ring()
_INV = [0] * N_DEV
for _k, _l in enumerate(_RING):
    _INV[_l] = _k


def kernel(x, w_mat, scale_x, scale_w):
    m_total, k_per = x.shape
    _, n = w_mat.shape
    m_per = m_total // N_DEV
    nh = n // 2

    ring = jnp.asarray(_RING, jnp.int32)
    inv = jnp.asarray(_INV, jnp.int32)
    pos = inv[lax.axis_index("i")]
    nbrs = jnp.stack([ring[(pos + 1) % N_DEV],
                      ring[(pos - 1) % N_DEV]]).astype(jnp.int32)
    sched_r = ring[(pos + R_STEPS - jnp.arange(R_STEPS)) % N_DEV]
    sched_l = ring[(pos - L_STEPS + jnp.arange(L_STEPS)) % N_DEV]

    def body(x_ref, w_ref, sx_ref, sw_ref, nbr_ref, schr_ref, schl_ref,
             out_ref, w_bf,
             send_ra, send_rb, send_la, send_lb,
             recv_ra, recv_rb, recv_la, recv_lb,
             ssem, rsem_ra, rsem_rb, rsem_la, rsem_lb):
        right = nbr_ref[0]
        left = nbr_ref[1]

        barrier_sem = pltpu.get_barrier_semaphore()
        for nbr in (left, right):
            pl.semaphore_signal(
                barrier_sem, inc=1,
                device_id=(nbr,), device_id_type=pl.DeviceIdType.MESH,
            )
        pl.semaphore_wait(barrier_sem, 2)

        w_bf[...] = w_ref[...].astype(jnp.bfloat16)

        def partial_chunk(c):
            xa = x_ref[pl.ds(c * m_per, m_per), :].astype(jnp.bfloat16)
            return jnp.dot(xa, w_bf[...],
                           preferred_element_type=jnp.float32)

        flows = {
            "ra": (send_ra, recv_ra, rsem_ra, 0, right, R_STEPS, 0),
            "rb": (send_rb, recv_rb, rsem_rb, 1, right, R_STEPS, nh),
            "la": (send_la, recv_la, rsem_la, 2, left, L_STEPS, 0),
            "lb": (send_lb, recv_lb, rsem_lb, 3, left, L_STEPS, nh),
        }

        def mk(f, s):
            sbuf, rbuf, rsems, si, dev, _, _ = flows[f]
            return pltpu.make_async_remote_copy(
                src_ref=sbuf, dst_ref=rbuf.at[s % 2],
                send_sem=ssem.at[si], recv_sem=rsems.at[s],
                device_id=(dev,), device_id_type=pl.DeviceIdType.MESH)

        rdma = {f: [None] * flows[f][5] for f in flows}

        p_r = partial_chunk(schr_ref[0]).astype(jnp.bfloat16)
        p_l = partial_chunk(schl_ref[0]).astype(jnp.bfloat16)
        for f, p in (("ra", p_r), ("la", p_l), ("rb", p_r), ("lb", p_l)):
            sbuf = flows[f][0]
            lo = flows[f][6]
            sbuf[...] = p[:, lo:lo + nh]
            rdma[f][0] = mk(f, 0)
            rdma[f][0].start()

        for s in range(1, R_STEPS):
            p_r = partial_chunk(schr_ref[s]).astype(jnp.bfloat16)
            p_l = (partial_chunk(schl_ref[s]).astype(jnp.bfloat16)
                   if s < L_STEPS else None)
            for f, p in (("ra", p_r), ("la", p_l), ("rb", p_r), ("lb", p_l)):
                if flows[f][5] <= s:
                    continue
                sbuf, rbuf, _, _, _, _, lo = flows[f]
                prev = rdma[f][s - 1]
                prev.wait_send()
                prev.wait_recv()
                sbuf[...] = rbuf[(s - 1) % 2] + p[:, lo:lo + nh]
                rdma[f][s] = mk(f, s)
                rdma[f][s].start()

        p_own = partial_chunk(lax.axis_index("i"))
        scale = sx_ref[0] * sw_ref[0]
        for f in ("ra", "la"):
            last = flows[f][5] - 1
            rdma[f][last].wait_send()
            rdma[f][last].wait_recv()
        acc_a = (recv_ra[(R_STEPS - 1) % 2].astype(jnp.float32)
                 + recv_la[(L_STEPS - 1) % 2].astype(jnp.float32)
                 + p_own[:, :nh])
        y_a = acc_a * scale
        out_ref[:, 0:nh] = y_a * (1.0 / (1.0 + jnp.exp(-y_a)))
        for f in ("rb", "lb"):
            last = flows[f][5] - 1
            rdma[f][last].wait_send()
            rdma[f][last].wait_recv()
        acc_b = (recv_rb[(R_STEPS - 1) % 2].astype(jnp.float32)
                 + recv_lb[(L_STEPS - 1) % 2].astype(jnp.float32)
                 + p_own[:, nh:])
        y_b = acc_b * scale
        out_ref[:, nh:n] = y_b * (1.0 / (1.0 + jnp.exp(-y_b)))

    return pl.pallas_call(
        body,
        out_shape=jax.ShapeDtypeStruct((m_per, n), jnp.float32),
        in_specs=[
            pl.BlockSpec(memory_space=pltpu.VMEM),
            pl.BlockSpec(memory_space=pltpu.VMEM),
            pl.BlockSpec(memory_space=pltpu.SMEM),
            pl.BlockSpec(memory_space=pltpu.SMEM),
            pl.BlockSpec(memory_space=pltpu.SMEM),
            pl.BlockSpec(memory_space=pltpu.SMEM),
            pl.BlockSpec(memory_space=pltpu.SMEM),
        ],
        out_specs=pl.BlockSpec(memory_space=pltpu.VMEM),
        scratch_shapes=[
            pltpu.VMEM((k_per, n), jnp.bfloat16),
            pltpu.VMEM((m_per, nh), jnp.bfloat16),
            pltpu.VMEM((m_per, nh), jnp.bfloat16),
            pltpu.VMEM((m_per, nh), jnp.bfloat16),
            pltpu.VMEM((m_per, nh), jnp.bfloat16),
            pltpu.VMEM((2, m_per, nh), jnp.bfloat16),
            pltpu.VMEM((2, m_per, nh), jnp.bfloat16),
            pltpu.VMEM((2, m_per, nh), jnp.bfloat16),
            pltpu.VMEM((2, m_per, nh), jnp.bfloat16),
            pltpu.SemaphoreType.DMA((4,)),
            pltpu.SemaphoreType.DMA((R_STEPS,)),
            pltpu.SemaphoreType.DMA((R_STEPS,)),
            pltpu.SemaphoreType.DMA((L_STEPS,)),
            pltpu.SemaphoreType.DMA((L_STEPS,)),
        ],
        compiler_params=pltpu.CompilerParams(collective_id=0),
    )(x, w_mat, scale_x, scale_w, nbrs, sched_r, sched_l)


# device time: 105547 ns/iter; 1.0023x vs baseline; 1.0023x over previous
import jax
import jax.numpy as jnp
from jax import lax
from jax.experimental import pallas as pl
from jax.experimental.pallas import tpu as pltpu

N_DEV = 32
R_STEPS = 16
L_STEPS = 15


def _build_ring():
    coords = [(x, y, z) for z in range(4) for y in range(4) for x in range(2)]
    logical = {}
    l = 0
    for z in range(4):
        plane = sorted(c for c in coords if c[2] == z)
        for yi, y in enumerate(sorted({c[1] for c in plane})):
            for c in sorted((c for c in plane if c[1] == y),
                            reverse=bool(yi % 2)):
                logical[c] = l
                l += 1
    cyc = []
    for y in range(4):
        zs = range(4) if y % 2 == 0 else range(3, -1, -1)
        cyc += [(0, y, z) for z in zs]
    for y in range(3, -1, -1):
        zs = range(4) if y % 2 == 1 else range(3, -1, -1)
        cyc += [(1, y, z) for z in zs]
    assert len(set(cyc)) == N_DEV
    for a, b in zip(cyc, cyc[1:] + cyc[:1]):
        assert sum(abs(i - j) for i, j in zip(a, b)) == 1, (a, b)
    return [logical[c] for c in cyc]


_RING = _build_ring()
_INV = [0] * N_DEV
for _k, _l in enumerate(_RING):
    _INV[_l] = _k


def kernel(x, w_mat, scale_x, scale_w):
    m_total, k_per = x.shape
    _, n = w_mat.shape
    m_per = m_total // N_DEV
    nh = n // 2

    ring = jnp.asarray(_RING, jnp.int32)
    inv = jnp.asarray(_INV, jnp.int32)
    pos = inv[lax.axis_index("i")]
    nbrs = jnp.stack([ring[(pos + 1) % N_DEV],
                      ring[(pos - 1) % N_DEV]]).astype(jnp.int32)
    sched_r = ring[(pos + R_STEPS - jnp.arange(R_STEPS)) % N_DEV]
    sched_l = ring[(pos - L_STEPS + jnp.arange(L_STEPS)) % N_DEV]

    def body(x_ref, w_ref, sx_ref, sw_ref, nbr_ref, schr_ref, schl_ref,
             out_ref, w_bf,
             send_ra, send_rb, send_la, send_lb,
             recv_ra, recv_rb, recv_la, recv_lb,
             ssem, rsem_ra, rsem_rb, rsem_la, rsem_lb):
        right = nbr_ref[0]
        left = nbr_ref[1]

        barrier_sem = pltpu.get_barrier_semaphore()
        for nbr in (left, right):
            pl.semaphore_signal(
                barrier_sem, inc=1,
                device_id=(nbr,), device_id_type=pl.DeviceIdType.MESH,
            )

        w_bf[...] = w_ref[...].astype(jnp.bfloat16)

        def partial_chunk(c):
            xa = x_ref[pl.ds(c * m_per, m_per), :].astype(jnp.bfloat16)
            return jnp.dot(xa, w_bf[...],
                           preferred_element_type=jnp.float32)

        flows = {
            "ra": (send_ra, recv_ra, rsem_ra, 0, right, R_STEPS, 0),
            "rb": (send_rb, recv_rb, rsem_rb, 1, right, R_STEPS, nh),
            "la": (send_la, recv_la, rsem_la, 2, left, L_STEPS, 0),
            "lb": (send_lb, recv_lb, rsem_lb, 3, left, L_STEPS, nh),
        }

        def mk(f, s):
            sbuf, rbuf, rsems, si, dev, _, _ = flows[f]
            return pltpu.make_async_remote_copy(
                src_ref=sbuf, dst_ref=rbuf.at[s % 2],
                send_sem=ssem.at[si], recv_sem=rsems.at[s],
                device_id=(dev,), device_id_type=pl.DeviceIdType.MESH)

        rdma = {f: [None] * flows[f][5] for f in flows}

        p_r = partial_chunk(schr_ref[0]).astype(jnp.bfloat16)
        p_l = partial_chunk(schl_ref[0]).astype(jnp.bfloat16)
        for f, p in (("ra", p_r), ("la", p_l), ("rb", p_r), ("lb", p_l)):
            sbuf = flows[f][0]
            lo = flows[f][6]
            sbuf[...] = p[:, lo:lo + nh]
        pl.semaphore_wait(barrier_sem, 2)
        for f in ("ra", "la", "rb", "lb"):
            rdma[f][0] = mk(f, 0)
            rdma[f][0].start()

        for s in range(1, R_STEPS):
            p_r = partial_chunk(schr_ref[s]).astype(jnp.bfloat16)
            p_l = (partial_chunk(schl_ref[s]).astype(jnp.bfloat16)
                   if s < L_STEPS else None)
            for f, p in (("ra", p_r), ("la", p_l), ("rb", p_r), ("lb", p_l)):
                if flows[f][5] <= s:
                    continue
                sbuf, rbuf, _, _, _, _, lo = flows[f]
                prev = rdma[f][s - 1]
                prev.wait_send()
                prev.wait_recv()
                sbuf[...] = rbuf[(s - 1) % 2] + p[:, lo:lo + nh]
                rdma[f][s] = mk(f, s)
                rdma[f][s].start()

        p_own = partial_chunk(lax.axis_index("i"))
        scale = sx_ref[0] * sw_ref[0]
        for f in ("ra", "la"):
            last = flows[f][5] - 1
            rdma[f][last].wait_send()
            rdma[f][last].wait_recv()
        acc_a = (recv_ra[(R_STEPS - 1) % 2].astype(jnp.float32)
                 + recv_la[(L_STEPS - 1) % 2].astype(jnp.float32)
                 + p_own[:, :nh])
        y_a = acc_a * scale
        out_ref[:, 0:nh] = y_a * (1.0 / (1.0 + jnp.exp(-y_a)))
        for f in ("rb", "lb"):
            last = flows[f][5] - 1
            rdma[f][last].wait_send()
            rdma[f][last].wait_recv()
        acc_b = (recv_rb[(R_STEPS - 1) % 2].astype(jnp.float32)
                 + recv_lb[(L_STEPS - 1) % 2].astype(jnp.float32)
                 + p_own[:, nh:])
        y_b = acc_b * scale
        out_ref[:, nh:n] = y_b * (1.0 / (1.0 + jnp.exp(-y_b)))

    return pl.pallas_call(
        body,
        out_shape=jax.ShapeDtypeStruct((m_per, n), jnp.float32),
        in_specs=[
            pl.BlockSpec(memory_space=pltpu.VMEM),
            pl.BlockSpec(memory_space=pltpu.VMEM),
            pl.BlockSpec(memory_space=pltpu.SMEM),
            pl.BlockSpec(memory_space=pltpu.SMEM),
            pl.BlockSpec(memory_space=pltpu.SMEM),
            pl.BlockSpec(memory_space=pltpu.SMEM),
            pl.BlockSpec(memory_space=pltpu.SMEM),
        ],
        out_specs=pl.BlockSpec(memory_space=pltpu.VMEM),
        scratch_shapes=[
            pltpu.VMEM((k_per, n), jnp.bfloat16),
            pltpu.VMEM((m_per, nh), jnp.bfloat16),
            pltpu.VMEM((m_per, nh), jnp.bfloat16),
            pltpu.VMEM((m_per, nh), jnp.bfloat16),
            pltpu.VMEM((m_per, nh), jnp.bfloat16),
            pltpu.VMEM((2, m_per, nh), jnp.bfloat16),
            pltpu.VMEM((2, m_per, nh), jnp.bfloat16),
            pltpu.VMEM((2, m_per, nh), jnp.bfloat16),
            pltpu.VMEM((2, m_per, nh), jnp.bfloat16),
            pltpu.SemaphoreType.DMA((4,)),
            pltpu.SemaphoreType.DMA((R_STEPS,)),
            pltpu.SemaphoreType.DMA((R_STEPS,)),
            pltpu.SemaphoreType.DMA((L_STEPS,)),
            pltpu.SemaphoreType.DMA((L_STEPS,)),
        ],
        compiler_params=pltpu.CompilerParams(collective_id=0),
    )(x, w_mat, scale_x, scale_w, nbrs, sched_r, sched_l)
